# baseline (speedup 1.0000x reference)
import functools

import jax
import jax.numpy as jnp
from jax import lax
from jax.experimental import pallas as pl
from jax.experimental.pallas import tpu as pltpu
from jax.experimental.pallas import tpu_sc as plsc

_LANES = 16
_UNROLL = 8
_CUNROLL = 8


def _rsqrt(p):
    i = lax.bitcast_convert_type(p, jnp.int32)
    i = jnp.int32(0x5F3759DF) - (i >> 1)
    y = lax.bitcast_convert_type(i, jnp.float32)
    ph = p * 0.5
    y = y * (1.5 - ph * y * y)
    y = y * (1.5 - ph * y * y)
    return y


def _sc_body(T, L, NC, NW, RW,
             tb_h, lig_h, map_h, crd_h, x0_h, x1_h, x2_h, g0_h, g1_h, g2_h,
             out_n, out_d,
             map_v, crd_v, lig_v, act_v,
             x0_v, x1_v, x2_v, g0_v, g1_v, g2_v,
             bondrow_v, cidx_v, ncidx_v, mapnc_v, nbuf_v, dbuf_v):
    cid = lax.axis_index("c")
    sid = lax.axis_index("s")
    wid = sid * NC + cid

    pltpu.sync_copy(map_h, map_v.at[pl.ds(0, L)])
    pltpu.sync_copy(crd_h, crd_v)
    pltpu.sync_copy(lig_h, lig_v)
    pltpu.sync_copy(x0_h, x0_v.at[pl.ds(0, L)])
    pltpu.sync_copy(x1_h, x1_v.at[pl.ds(0, L)])
    pltpu.sync_copy(x2_h, x2_v.at[pl.ds(0, L)])
    pltpu.sync_copy(g0_h, g0_v.at[pl.ds(0, L)])
    pltpu.sync_copy(g1_h, g1_v.at[pl.ds(0, L)])
    pltpu.sync_copy(g2_h, g2_v.at[pl.ds(0, L)])

    nchunks = L // _LANES
    lane_iota = lax.iota(jnp.int32, _LANES)
    zero16 = jnp.zeros((_LANES,), jnp.float32)

    pad = pl.ds(L, _LANES)
    x0_v[pad] = zero16
    x1_v[pad] = zero16
    x2_v[pad] = zero16
    big = jnp.full((_LANES,), 1.0e9, jnp.float32)
    g0_v[pad] = big
    g1_v[pad] = big
    g2_v[pad] = big

    bondrow_v[pl.ds(T, _LANES)] = zero16

    def _pre(c, cntn):
        sl = pl.ds(c * _LANES, _LANES)
        mp = map_v[sl]
        la = plsc.load_gather(lig_v, [mp])
        cr = crd_v[sl]
        act_v[sl] = la * cr
        m = (1.0 - la) * cr > 0.0
        cols = lane_iota + c * _LANES
        plsc.store_compressed(ncidx_v.at[pl.ds(cntn, _LANES)], cols, mask=m)
        plsc.store_compressed(mapnc_v.at[pl.ds(cntn, _LANES)], mp, mask=m)
        return cntn + plsc.all_reduce_population_count(m)[0]

    cnt_nc = lax.fori_loop(0, nchunks, _pre, jnp.int32(0), unroll=False)
    sent_t = jnp.full((_LANES,), T, jnp.int32)
    zero_i = jnp.zeros((_LANES,), jnp.int32)
    for k in range(_CUNROLL):
        mapnc_v[pl.ds(cnt_nc + k * _LANES, _LANES)] = sent_t
        ncidx_v[pl.ds(cnt_nc + k * _LANES, _LANES)] = zero_i
    ncw = _CUNROLL * _LANES
    n_cb = (cnt_nc + ncw - 1) // ncw

    def _recompute(t1, _cnt):
        pltpu.sync_copy(tb_h.at[t1], bondrow_v.at[pl.ds(0, T)])

        def _cb(c, cnt2):
            masks = []
            colss = []
            for k in range(_CUNROLL):
                sl = pl.ds((c * _CUNROLL + k) * _LANES, _LANES)
                bond = plsc.load_gather(bondrow_v, [mapnc_v[sl]])
                masks.append(bond > 0.0)
                colss.append(ncidx_v[sl])
            for k in range(_CUNROLL):
                plsc.store_compressed(cidx_v.at[pl.ds(cnt2, _LANES)],
                                      colss[k], mask=masks[k])
                cnt2 = cnt2 + plsc.all_reduce_population_count(masks[k])[0]
            return cnt2

        cnt = lax.fori_loop(0, n_cb, _cb, jnp.int32(0), unroll=False)
        sent = jnp.full((_LANES,), L, jnp.int32)
        for k in range(_UNROLL):
            cidx_v[pl.ds(cnt + k * _LANES, _LANES)] = sent
        return cnt

    def _row(rr, carry, rc):
        accs, t1_prev, cnt = carry
        a1 = (rc * NW + wid) * _LANES + rr
        asl = pl.ds(a1, _LANES)
        act = act_v[asl][0]

        def _active(args):
            accs, t1_prev, cnt = args
            t1 = map_v[asl][0]
            cnt = lax.cond(t1 != t1_prev, lambda c: _recompute(t1, c),
                           lambda c: c, cnt)
            xa = x0_v[asl][0]
            ya = x1_v[asl][0]
            za = x2_v[asl][0]
            gxa = g0_v[asl][0]
            gya = g1_v[asl][0]
            gza = g2_v[asl][0]

            def _chunk(c, accs2):
                out = []
                for k in range(_UNROLL):
                    nv, dv = accs2[k]
                    sl = pl.ds((c * _UNROLL + k) * _LANES, _LANES)
                    idx = cidx_v[sl]
                    dx = plsc.load_gather(x0_v, [idx]) - xa
                    dy = plsc.load_gather(x1_v, [idx]) - ya
                    dz = plsc.load_gather(x2_v, [idx]) - za
                    pd2 = dx * dx + dy * dy + dz * dz
                    ex = plsc.load_gather(g0_v, [idx]) - gxa
                    ey = plsc.load_gather(g1_v, [idx]) - gya
                    ez = plsc.load_gather(g2_v, [idx]) - gza
                    gd2 = ex * ex + ey * ey + ez * ez
                    p = pd2 * gd2
                    sq = p * _rsqrt(p)
                    term = pd2 + gd2 - 2.0 * sq
                    m = jnp.where(gd2 < 5.76, 1.0, 0.0)
                    out.append((nv + term * m, dv + m))
                return tuple(out)

            nch = (cnt + (_UNROLL * _LANES - 1)) // (_UNROLL * _LANES)
            accs = lax.fori_loop(0, nch, _chunk, accs, unroll=False)
            return accs, t1, cnt

        return lax.cond(act > 0.0, _active, lambda a: a,
                        (accs, t1_prev, cnt))

    def _rowchunk(rc, carry):
        return lax.fori_loop(0, _LANES, functools.partial(_row, rc=rc),
                             carry, unroll=False)

    accs0 = tuple((zero16, zero16) for _ in range(_UNROLL))
    accs, _, _ = lax.fori_loop(0, RW // _LANES, _rowchunk,
                               (accs0, jnp.int32(-1), jnp.int32(0)),
                               unroll=False)

    numv = accs[0][0]
    denv = accs[0][1]
    for k in range(1, _UNROLL):
        numv = numv + accs[k][0]
        denv = denv + accs[k][1]
    nbuf_v[...] = numv
    dbuf_v[...] = denv
    pltpu.sync_copy(nbuf_v, out_n.at[wid])
    pltpu.sync_copy(dbuf_v, out_d.at[wid])


def _reduce_body(n_ref, d_ref, out_w, out_l):
    num = jnp.sum(n_ref[...])
    den = jnp.sum(d_ref[...])
    loss = num / jnp.maximum(den, 1.0)
    out_w[...] = jnp.full((1, 1), loss, dtype=jnp.float32)
    out_l[...] = jnp.full((1, 1), loss, dtype=jnp.float32)


def kernel(is_ligand, token_bonds, atom_to_token_map, crd_mask_L, X_L, X_gt_L):
    T = is_ligand.shape[0]
    L = atom_to_token_map.shape[0]
    NC, NS = 2, 16
    NW = NC * NS
    RW = L // NW

    tb_f = token_bonds.astype(jnp.float32)
    lig_f = is_ligand.astype(jnp.float32)
    map_i = atom_to_token_map.astype(jnp.int32)
    crd_f = crd_mask_L[0].astype(jnp.float32)
    x0, x1, x2 = (X_L[0, :, k] for k in range(3))
    g0, g1, g2 = (X_gt_L[0, :, k] for k in range(3))

    mesh = plsc.VectorSubcoreMesh(core_axis_name="c", subcore_axis_name="s",
                                  num_cores=NC, num_subcores=NS)
    sc = pl.kernel(
        functools.partial(_sc_body, T, L, NC, NW, RW),
        out_type=[jax.ShapeDtypeStruct((NW, _LANES), jnp.float32),
                  jax.ShapeDtypeStruct((NW, _LANES), jnp.float32)],
        mesh=mesh,
        compiler_params=pltpu.CompilerParams(needs_layout_passes=False),
        scratch_types=[
            pltpu.VMEM((L + _LANES,), jnp.int32),
            pltpu.VMEM((L,), jnp.float32),
            pltpu.VMEM((T,), jnp.float32),
            pltpu.VMEM((L + _LANES,), jnp.float32),
            pltpu.VMEM((L + _LANES,), jnp.float32),
            pltpu.VMEM((L + _LANES,), jnp.float32),
            pltpu.VMEM((L + _LANES,), jnp.float32),
            pltpu.VMEM((L + _LANES,), jnp.float32),
            pltpu.VMEM((L + _LANES,), jnp.float32),
            pltpu.VMEM((L + _LANES,), jnp.float32),
            pltpu.VMEM((T + _LANES,), jnp.float32),
            pltpu.VMEM((L + _UNROLL * _LANES,), jnp.int32),
            pltpu.VMEM((L + _CUNROLL * _LANES,), jnp.int32),
            pltpu.VMEM((L + _CUNROLL * _LANES,), jnp.int32),
            pltpu.VMEM((_LANES,), jnp.float32),
            pltpu.VMEM((_LANES,), jnp.float32),
        ],
    )
    part_n, part_d = sc(tb_f, lig_f, map_i, crd_f, x0, x1, x2, g0, g1, g2)

    out_w, out_l = pl.pallas_call(
        _reduce_body,
        out_shape=[jax.ShapeDtypeStruct((1, 1), jnp.float32),
                   jax.ShapeDtypeStruct((1, 1), jnp.float32)],
    )(part_n, part_d)
    return (out_w.reshape(()), out_l.reshape(()))


# speedup vs baseline: 1.0644x; 1.0644x over previous
import functools

import jax
import jax.numpy as jnp
from jax import lax
from jax.experimental import pallas as pl
from jax.experimental.pallas import tpu as pltpu
from jax.experimental.pallas import tpu_sc as plsc

_LANES = 16
_UNROLL = 8
_CUNROLL = 8


def _rsqrt(p):
    i = lax.bitcast_convert_type(p, jnp.int32)
    i = jnp.int32(0x5F3759DF) - (i >> 1)
    y = lax.bitcast_convert_type(i, jnp.float32)
    ph = p * 0.5
    y = y * (1.5 - ph * y * y)
    y = y * (1.5 - ph * y * y)
    return y


def _sc_body(T, L, NC, NW, RW,
             tb_h, lig_h, map_h, crd_h, xg_h,
             out_n, out_d,
             map_v, crd_v, lig_v, act_v, xg_v,
             bondrow_v, cidx_v, ncidx_v, mapnc_v, nbuf_v, dbuf_v):
    cid = lax.axis_index("c")
    sid = lax.axis_index("s")
    wid = sid * NC + cid

    pltpu.sync_copy(map_h, map_v.at[pl.ds(0, L)])
    pltpu.sync_copy(crd_h, crd_v)
    pltpu.sync_copy(lig_h, lig_v)
    pltpu.sync_copy(xg_h, xg_v)

    LP = L + _LANES
    x0_v = xg_v.at[pl.ds(0 * LP, LP)]
    x1_v = xg_v.at[pl.ds(1 * LP, LP)]
    x2_v = xg_v.at[pl.ds(2 * LP, LP)]
    g0_v = xg_v.at[pl.ds(3 * LP, LP)]
    g1_v = xg_v.at[pl.ds(4 * LP, LP)]
    g2_v = xg_v.at[pl.ds(5 * LP, LP)]

    nchunks = L // _LANES
    lane_iota = lax.iota(jnp.int32, _LANES)
    zero16 = jnp.zeros((_LANES,), jnp.float32)

    bondrow_v[pl.ds(T, _LANES)] = zero16

    def _pre(c, cntn):
        sl = pl.ds(c * _LANES, _LANES)
        mp = map_v[sl]
        la = plsc.load_gather(lig_v, [mp])
        cr = crd_v[sl]
        act_v[sl] = la * cr
        m = (1.0 - la) * cr > 0.0
        cols = lane_iota + c * _LANES
        plsc.store_compressed(ncidx_v.at[pl.ds(cntn, _LANES)], cols, mask=m)
        plsc.store_compressed(mapnc_v.at[pl.ds(cntn, _LANES)], mp, mask=m)
        return cntn + plsc.all_reduce_population_count(m)[0]

    cnt_nc = lax.fori_loop(0, nchunks, _pre, jnp.int32(0), unroll=False)
    sent_t = jnp.full((_LANES,), T, jnp.int32)
    zero_i = jnp.zeros((_LANES,), jnp.int32)
    for k in range(_CUNROLL):
        mapnc_v[pl.ds(cnt_nc + k * _LANES, _LANES)] = sent_t
        ncidx_v[pl.ds(cnt_nc + k * _LANES, _LANES)] = zero_i
    ncw = _CUNROLL * _LANES
    n_cb = (cnt_nc + ncw - 1) // ncw

    def _recompute(t1, _cnt):
        pltpu.sync_copy(tb_h.at[t1], bondrow_v.at[pl.ds(0, T)])

        def _cb(c, cnt2):
            masks = []
            colss = []
            for k in range(_CUNROLL):
                sl = pl.ds((c * _CUNROLL + k) * _LANES, _LANES)
                bond = plsc.load_gather(bondrow_v, [mapnc_v[sl]])
                masks.append(bond > 0.0)
                colss.append(ncidx_v[sl])
            for k in range(_CUNROLL):
                plsc.store_compressed(cidx_v.at[pl.ds(cnt2, _LANES)],
                                      colss[k], mask=masks[k])
                cnt2 = cnt2 + plsc.all_reduce_population_count(masks[k])[0]
            return cnt2

        cnt = lax.fori_loop(0, n_cb, _cb, jnp.int32(0), unroll=False)
        sent = jnp.full((_LANES,), L, jnp.int32)
        for k in range(_UNROLL):
            cidx_v[pl.ds(cnt + k * _LANES, _LANES)] = sent
        return cnt

    def _row(rr, carry, rc):
        accs, t1_prev, cnt = carry
        a1 = (wid * (RW // _LANES) + rc) * _LANES + rr
        asl = pl.ds(a1, _LANES)
        act = act_v[asl][0]

        def _active(args):
            accs, t1_prev, cnt = args
            t1 = map_v[asl][0]
            cnt = lax.cond(t1 != t1_prev, lambda c: _recompute(t1, c),
                           lambda c: c, cnt)
            xa = x0_v[asl][0]
            ya = x1_v[asl][0]
            za = x2_v[asl][0]
            gxa = g0_v[asl][0]
            gya = g1_v[asl][0]
            gza = g2_v[asl][0]

            def _chunk(c, accs2):
                out = []
                for k in range(_UNROLL):
                    nv, dv = accs2[k]
                    sl = pl.ds((c * _UNROLL + k) * _LANES, _LANES)
                    idx = cidx_v[sl]
                    dx = plsc.load_gather(x0_v, [idx]) - xa
                    dy = plsc.load_gather(x1_v, [idx]) - ya
                    dz = plsc.load_gather(x2_v, [idx]) - za
                    pd2 = dx * dx + dy * dy + dz * dz
                    ex = plsc.load_gather(g0_v, [idx]) - gxa
                    ey = plsc.load_gather(g1_v, [idx]) - gya
                    ez = plsc.load_gather(g2_v, [idx]) - gza
                    gd2 = ex * ex + ey * ey + ez * ez
                    p = pd2 * gd2
                    sq = p * _rsqrt(p)
                    term = pd2 + gd2 - 2.0 * sq
                    m = jnp.where(gd2 < 5.76, 1.0, 0.0)
                    out.append((nv + term * m, dv + m))
                return tuple(out)

            nch = (cnt + (_UNROLL * _LANES - 1)) // (_UNROLL * _LANES)
            accs = lax.fori_loop(0, nch, _chunk, accs, unroll=False)
            return accs, t1, cnt

        return lax.cond(act > 0.0, _active, lambda a: a,
                        (accs, t1_prev, cnt))

    def _rowchunk(rc, carry):
        return lax.fori_loop(0, _LANES, functools.partial(_row, rc=rc),
                             carry, unroll=False)

    accs0 = tuple((zero16, zero16) for _ in range(_UNROLL))
    accs, _, _ = lax.fori_loop(0, RW // _LANES, _rowchunk,
                               (accs0, jnp.int32(-1), jnp.int32(0)),
                               unroll=False)

    numv = accs[0][0]
    denv = accs[0][1]
    for k in range(1, _UNROLL):
        numv = numv + accs[k][0]
        denv = denv + accs[k][1]
    nbuf_v[...] = numv
    dbuf_v[...] = denv
    pltpu.sync_copy(nbuf_v, out_n.at[wid])
    pltpu.sync_copy(dbuf_v, out_d.at[wid])


def _reduce_body(n_ref, d_ref, out_w, out_l):
    num = jnp.sum(n_ref[...])
    den = jnp.sum(d_ref[...])
    loss = num / jnp.maximum(den, 1.0)
    out_w[...] = jnp.full((1, 1), loss, dtype=jnp.float32)
    out_l[...] = jnp.full((1, 1), loss, dtype=jnp.float32)


def kernel(is_ligand, token_bonds, atom_to_token_map, crd_mask_L, X_L, X_gt_L):
    T = is_ligand.shape[0]
    L = atom_to_token_map.shape[0]
    NC, NS = 2, 16
    NW = NC * NS
    RW = L // NW

    tb_f = token_bonds.astype(jnp.float32)
    lig_f = is_ligand.astype(jnp.float32)
    map_i = atom_to_token_map.astype(jnp.int32)
    crd_f = crd_mask_L[0].astype(jnp.float32)
    pad0 = jnp.zeros((_LANES,), jnp.float32)
    padbig = jnp.full((_LANES,), 1.0e9, jnp.float32)
    xg = jnp.concatenate(
        [X_L[0, :, 0], pad0, X_L[0, :, 1], pad0, X_L[0, :, 2], pad0,
         X_gt_L[0, :, 0], padbig, X_gt_L[0, :, 1], padbig,
         X_gt_L[0, :, 2], padbig])

    mesh = plsc.VectorSubcoreMesh(core_axis_name="c", subcore_axis_name="s",
                                  num_cores=NC, num_subcores=NS)
    sc = pl.kernel(
        functools.partial(_sc_body, T, L, NC, NW, RW),
        out_type=[jax.ShapeDtypeStruct((NW, _LANES), jnp.float32),
                  jax.ShapeDtypeStruct((NW, _LANES), jnp.float32)],
        mesh=mesh,
        compiler_params=pltpu.CompilerParams(needs_layout_passes=False),
        scratch_types=[
            pltpu.VMEM((L + _LANES,), jnp.int32),
            pltpu.VMEM((L,), jnp.float32),
            pltpu.VMEM((T,), jnp.float32),
            pltpu.VMEM((L + _LANES,), jnp.float32),
            pltpu.VMEM((6 * (L + _LANES),), jnp.float32),
            pltpu.VMEM((T + _LANES,), jnp.float32),
            pltpu.VMEM((L + _UNROLL * _LANES,), jnp.int32),
            pltpu.VMEM((L + _CUNROLL * _LANES,), jnp.int32),
            pltpu.VMEM((L + _CUNROLL * _LANES,), jnp.int32),
            pltpu.VMEM((_LANES,), jnp.float32),
            pltpu.VMEM((_LANES,), jnp.float32),
        ],
    )
    part_n, part_d = sc(tb_f, lig_f, map_i, crd_f, xg)

    out_w, out_l = pl.pallas_call(
        _reduce_body,
        out_shape=[jax.ShapeDtypeStruct((1, 1), jnp.float32),
                   jax.ShapeDtypeStruct((1, 1), jnp.float32)],
    )(part_n, part_d)
    return (out_w.reshape(()), out_l.reshape(()))
